# Initial kernel scaffold; baseline (speedup 1.0000x reference)
import jax
import jax.numpy as jnp
from jax.experimental import pallas as pl


def kernel(context_features, example_features, mask):
    raise NotImplementedError("write your pallas kernel here")


# speedup vs baseline: 1.4198x; 1.4198x over previous
import functools

import jax
import jax.numpy as jnp
from jax import lax
from jax.experimental import pallas as pl
from jax.experimental.pallas import tpu as pltpu
from jax.experimental.pallas import tpu_sc as plsc

B, L, DC, DE = 16, 4096, 128, 128
NC, NS = 2, 16
NW = NC * NS
HALF = L // 2
BATCH = 128
NBATCH = HALF // BATCH


def _sc_body(ctx_hbm, ex_hbm, mask_hbm, octx_hbm, oex_hbm,
             mask_v, valid_v, idx_v, rows_v, rep_v, ctx_v, sem):
    c = lax.axis_index("c")
    s = lax.axis_index("s")
    wid = s * NC + c
    b = wid // 2
    h = wid % 2

    iota = lax.iota(jnp.int32, 16)

    pltpu.sync_copy(mask_hbm.at[b], mask_v)

    def comp_body(i, off):
        moff = pl.multiple_of(i * 16, 16)
        m = mask_v[pl.ds(moff, 16)]
        keys = iota + (1 - m) * 16
        _, pos_sorted = plsc.sort_key_val(keys, iota + i * 16)
        valid_v[pl.ds(off, 16)] = pos_sorted
        cntv = plsc.all_reduce_population_count(m > 0)
        return off + cntv[0]

    nv = lax.fori_loop(0, L // 16, comp_body, jnp.int32(0))

    @pl.when(nv == 0)
    def _():
        valid_v[pl.ds(0, 16)] = iota * 0
    nvc = jnp.maximum(nv, 1)
    nvb = jnp.broadcast_to(nvc, (16,))

    pltpu.sync_copy(ctx_hbm.at[b], ctx_v)
    chunks = [ctx_v[pl.ds(j * 16, 16)] for j in range(DC // 16)]

    def rep_body(i, _):
        for j in range(DC // 16):
            rep_v[i, pl.ds(j * 16, 16)] = chunks[j]
        return 0

    lax.fori_loop(0, BATCH, rep_body, 0)

    out_base = b * L + h * HALF

    def gath_body(g, _):
        lbase = h * HALF + g * BATCH
        for j in range(BATCH // 16):
            lvec = iota + (lbase + j * 16)
            lmod = lax.rem(lvec, nvb)
            colv = plsc.load_gather(valid_v, [lmod])
            idx_v[pl.ds(j * 16, 16)] = colv + b * L
        pltpu.async_copy(ex_hbm.at[idx_v], rows_v, sem).wait()
        row0 = out_base + g * BATCH
        pltpu.sync_copy(rows_v, oex_hbm.at[pl.ds(row0, BATCH)])
        pltpu.sync_copy(rep_v, octx_hbm.at[pl.ds(row0, BATCH)])
        return 0

    lax.fori_loop(0, NBATCH, gath_body, 0)


_flatten_sc = functools.partial(
    pl.kernel,
    out_type=(
        jax.ShapeDtypeStruct((B * L, DC), jnp.float32),
        jax.ShapeDtypeStruct((B * L, DE), jnp.float32),
    ),
    mesh=plsc.VectorSubcoreMesh(core_axis_name="c", subcore_axis_name="s"),
    compiler_params=pltpu.CompilerParams(needs_layout_passes=False),
    scratch_types=[
        pltpu.VMEM((L,), jnp.int32),
        pltpu.VMEM((L + 16,), jnp.int32),
        pltpu.VMEM((BATCH,), jnp.int32),
        pltpu.VMEM((BATCH, DE), jnp.float32),
        pltpu.VMEM((BATCH, DC), jnp.float32),
        pltpu.VMEM((DC,), jnp.float32),
        pltpu.SemaphoreType.DMA,
    ],
)(_sc_body)


def kernel(context_features, example_features, mask):
    ex_flat = example_features.reshape(B * L, DE)
    mask_i = mask.astype(jnp.int32)
    flat_ctx, flat_ex = _flatten_sc(context_features, ex_flat, mask_i)
    return flat_ctx, flat_ex


# speedup vs baseline: 1.6001x; 1.1270x over previous
import functools

import jax
import jax.numpy as jnp
from jax import lax
from jax.experimental import pallas as pl
from jax.experimental.pallas import tpu as pltpu
from jax.experimental.pallas import tpu_sc as plsc

B, L, DC, DE = 16, 4096, 128, 128
NC, NS = 2, 16
NW = NC * NS
HALF = L // 2
BATCH = 128
NBATCH = HALF // BATCH
REP = 256
NREP = HALF // REP


def _sc_body(ctx_hbm, ex_hbm, mask_hbm, octx_hbm, oex_hbm,
             mask_v, valid_v, idx_v, rows_v, rep_v, ctx_v,
             sem_m, sem_c, sem_g0, sem_g1, sem_w0, sem_w1):
    c = lax.axis_index("c")
    s = lax.axis_index("s")
    wid = s * NC + c
    b = wid // 2
    h = wid % 2
    out_base = b * L + h * HALF

    iota = lax.iota(jnp.int32, 16)

    mask_dma = pltpu.async_copy(mask_hbm.at[b], mask_v, sem_m)
    pltpu.sync_copy(ctx_hbm.at[b], ctx_v)

    chunks = [ctx_v[pl.ds(j * 16, 16)] for j in range(DC // 16)]

    def rep_body(i, _):
        for j in range(DC // 16):
            rep_v[i, pl.ds(j * 16, 16)] = chunks[j]
        return 0

    lax.fori_loop(0, REP, rep_body, 0)
    ctx_descs = [
        pltpu.async_copy(rep_v, octx_hbm.at[pl.ds(out_base + r * REP, REP)],
                         sem_c)
        for r in range(NREP)
    ]

    mask_dma.wait()

    def comp_body(i, off):
        moff = pl.multiple_of(i * 16, 16)
        m = mask_v[pl.ds(moff, 16)]
        keys = iota + (1 - m) * 16
        _, pos_sorted = plsc.sort_key_val(keys, iota + i * 16)
        valid_v[pl.ds(off, 16)] = pos_sorted
        cntv = plsc.all_reduce_population_count(m > 0)
        return off + cntv[0]

    nv = lax.fori_loop(0, L // 16, comp_body, jnp.int32(0))

    @pl.when(nv == 0)
    def _():
        valid_v[pl.ds(0, 16)] = iota * 0

    nvb = jnp.broadcast_to(jnp.maximum(nv, 1), (16,))

    def build_idx(g):
        lbase = h * HALF + g * BATCH
        slot = g % 2
        for j in range(BATCH // 16):
            lvec = iota + (lbase + j * 16)
            colv = plsc.load_gather(valid_v, [lax.rem(lvec, nvb)])
            idx_v[slot, pl.ds(j * 16, 16)] = colv + b * L

    sem_g = (sem_g0, sem_g1)
    sem_w = (sem_w0, sem_w1)
    gather_descs = [None, None]
    write_descs = [None, None]

    build_idx(0)
    gather_descs[0] = pltpu.async_copy(ex_hbm.at[idx_v.at[0]], rows_v.at[0],
                                       sem_g[0])
    for g in range(NBATCH):
        slot = g % 2
        nxt = 1 - slot
        if g + 1 < NBATCH:
            build_idx(g + 1)
            if write_descs[nxt] is not None:
                write_descs[nxt].wait()
            gather_descs[nxt] = pltpu.async_copy(
                ex_hbm.at[idx_v.at[nxt]], rows_v.at[nxt], sem_g[nxt])
        gather_descs[slot].wait()
        write_descs[slot] = pltpu.async_copy(
            rows_v.at[slot], oex_hbm.at[pl.ds(out_base + g * BATCH, BATCH)],
            sem_w[slot])

    write_descs[0].wait()
    write_descs[1].wait()
    for d in ctx_descs:
        d.wait()


_flatten_sc = functools.partial(
    pl.kernel,
    out_type=(
        jax.ShapeDtypeStruct((B * L, DC), jnp.float32),
        jax.ShapeDtypeStruct((B * L, DE), jnp.float32),
    ),
    mesh=plsc.VectorSubcoreMesh(core_axis_name="c", subcore_axis_name="s"),
    compiler_params=pltpu.CompilerParams(needs_layout_passes=False),
    scratch_types=[
        pltpu.VMEM((L,), jnp.int32),
        pltpu.VMEM((L + 16,), jnp.int32),
        pltpu.VMEM((2, BATCH), jnp.int32),
        pltpu.VMEM((2, BATCH, DE), jnp.float32),
        pltpu.VMEM((REP, DC), jnp.float32),
        pltpu.VMEM((DC,), jnp.float32),
        pltpu.SemaphoreType.DMA,
        pltpu.SemaphoreType.DMA,
        pltpu.SemaphoreType.DMA,
        pltpu.SemaphoreType.DMA,
        pltpu.SemaphoreType.DMA,
        pltpu.SemaphoreType.DMA,
    ],
)(_sc_body)


def kernel(context_features, example_features, mask):
    ex_flat = example_features.reshape(B * L, DE)
    mask_i = mask.astype(jnp.int32)
    flat_ctx, flat_ex = _flatten_sc(context_features, ex_flat, mask_i)
    return flat_ctx, flat_ex
